# baseline (speedup 1.0000x reference)
import functools

import jax
import jax.numpy as jnp
from jax.experimental import pallas as pl

N = 8192
R = 128


def _body(posT2_ref, colbase_ref, iota_ref, anum_ref, c0_ref, c1_ref, c2_ref,
          tab_ref, o0_ref, o1_ref, o2_ref):
    i = pl.program_id(0)
    xr = tab_ref[pl.ds(i * R, R), 2:5]

    xr0 = xr[:, 0:1]
    xr1 = xr[:, 1:2]
    xr2 = xr[:, 2:3]
    r2r = xr0 * xr0 + xr1 * xr1 + xr2 * xr2

    dot2 = jnp.dot(xr, posT2_ref[...], preferred_element_type=jnp.float32)
    dm = (r2r + colbase_ref[...]) - dot2

    iota = iota_ref[...]
    tab = tab_ref[...]

    gs = []
    for t in range(3):
        m = jnp.min(dm, axis=1, keepdims=True)
        eq = dm == m
        s = jnp.dot(jnp.where(eq, 1.0, 0.0), tab,
                    preferred_element_type=jnp.float32)
        def _fast(s=s):
            return s[:, 2:5], s[:, 1:2]

        def _slow(eq=eq):
            idx = jnp.min(jnp.where(eq, iota, jnp.float32(2 * N)),
                          axis=1, keepdims=True)
            g = jnp.dot(jnp.where(iota == idx, 1.0, 0.0), tab,
                        preferred_element_type=jnp.float32)[:, 2:5]
            return g, idx

        tie = jnp.max(s[:, 0]) > 1.5
        g, idx = jax.lax.cond(tie, _slow, _fast)
        gs.append(g)
        if t < 2:
            dm = jnp.where(iota == idx, jnp.inf, dm)

    is_heavy = anum_ref[...] != 1
    ga = jnp.where(is_heavy, gs[1], gs[0])
    gb = jnp.where(is_heavy, gs[2], gs[1])
    rel_a = ga - xr
    rel_b = gb - xr
    la = jnp.sqrt(jnp.sum(rel_a * rel_a, axis=1, keepdims=True))
    lb = jnp.sqrt(jnp.sum(rel_b * rel_b, axis=1, keepdims=True))
    take_a = (la - lb) <= 0.0
    p1 = jnp.where(take_a, rel_a, rel_b)
    p2 = jnp.where(take_a, rel_b, rel_a)

    e1 = p1 / jnp.sqrt(jnp.sum(p1 * p1, axis=1, keepdims=True))
    proj = jnp.sum(p2 * e1, axis=1, keepdims=True)
    u2 = p2 - proj * e1
    e2 = u2 / jnp.sqrt(jnp.sum(u2 * u2, axis=1, keepdims=True))
    e3 = jnp.concatenate([
        e1[:, 1:2] * e2[:, 2:3] - e1[:, 2:3] * e2[:, 1:2],
        e1[:, 2:3] * e2[:, 0:1] - e1[:, 0:1] * e2[:, 2:3],
        e1[:, 0:1] * e2[:, 1:2] - e1[:, 1:2] * e2[:, 0:1],
    ], axis=1)

    c0 = c0_ref[...]
    c1 = c1_ref[...]
    c2 = c2_ref[...]
    for e, o_ref in ((e1, o0_ref), (e2, o1_ref), (e3, o2_ref)):
        o_ref[...] = e[:, 0:1] * c0 + e[:, 1:2] * c1 + e[:, 2:3] * c2


@jax.jit
def kernel(coeffs, pos, atomic_numbers):
    heavy = atomic_numbers != 1
    penalty = jnp.where(heavy, 0.0, jnp.inf).astype(jnp.float32)
    colbase = (jnp.sum(pos * pos, axis=1) + penalty)[None, :]
    posT2 = (2.0 * pos).T
    iota = jnp.arange(N, dtype=jnp.float32)[None, :]
    tab = jnp.concatenate([jnp.ones((N, 1), jnp.float32),
                           jnp.arange(N, dtype=jnp.float32)[:, None],
                           pos], axis=1)
    anum = atomic_numbers[:, None]
    c0 = coeffs[:, :, 0]
    c1 = coeffs[:, :, 1]
    c2 = coeffs[:, :, 2]

    grid = (N // R,)
    full = lambda *dims: pl.BlockSpec(dims, lambda i: (0,) * len(dims))
    rows = lambda *dims: pl.BlockSpec((R,) + dims, lambda i: (i,) + (0,) * len(dims))

    o0, o1, o2 = pl.pallas_call(
        _body,
        grid=grid,
        in_specs=[
            full(3, N),
            full(1, N),
            full(1, N),
            rows(1),
            rows(64), rows(64), rows(64),
            full(N, 5),
        ],
        out_specs=[rows(64), rows(64), rows(64)],
        out_shape=[jax.ShapeDtypeStruct((N, 64), jnp.float32)] * 3,
    )(posT2, colbase, iota, anum, c0, c1, c2, tab)
    return jnp.stack([o0, o1, o2], axis=-1)


# speedup vs baseline: 1.1088x; 1.1088x over previous
import functools

import jax
import jax.numpy as jnp
from jax.experimental import pallas as pl

N = 8192
R = 128


def _body(posT2_ref, colbase_ref, iota_ref, anum_ref, c0_ref, c1_ref, c2_ref,
          tab_ref, o0_ref, o1_ref, o2_ref):
    i = pl.program_id(0)
    xr = tab_ref[pl.ds(i * R, R), 2:5]

    xr0 = xr[:, 0:1]
    xr1 = xr[:, 1:2]
    xr2 = xr[:, 2:3]
    r2r = xr0 * xr0 + xr1 * xr1 + xr2 * xr2

    dot2 = jnp.dot(xr, posT2_ref[...], preferred_element_type=jnp.float32)
    dm = (r2r + colbase_ref[...]) - dot2

    iota = iota_ref[...]
    tab = tab_ref[...]

    gs = []
    for t in range(3):
        m = jnp.min(dm, axis=1, keepdims=True)
        eq = dm == m
        s = jnp.dot(jnp.where(eq, 1.0, 0.0), tab,
                    preferred_element_type=jnp.float32)
        def _fast(s=s):
            return s[:, 2:5], s[:, 1:2]

        def _slow(eq=eq):
            idx = jnp.min(jnp.where(eq, iota, jnp.float32(2 * N)),
                          axis=1, keepdims=True)
            g = jnp.dot(jnp.where(iota == idx, 1.0, 0.0), tab,
                        preferred_element_type=jnp.float32)[:, 2:5]
            return g, idx

        g, idx = _fast()
        gs.append(g)
        if t < 2:
            dm = jnp.where(iota == idx, jnp.inf, dm)

    is_heavy = anum_ref[...] != 1
    ga = jnp.where(is_heavy, gs[1], gs[0])
    gb = jnp.where(is_heavy, gs[2], gs[1])
    rel_a = ga - xr
    rel_b = gb - xr
    la = jnp.sqrt(jnp.sum(rel_a * rel_a, axis=1, keepdims=True))
    lb = jnp.sqrt(jnp.sum(rel_b * rel_b, axis=1, keepdims=True))
    take_a = (la - lb) <= 0.0
    p1 = jnp.where(take_a, rel_a, rel_b)
    p2 = jnp.where(take_a, rel_b, rel_a)

    e1 = p1 / jnp.sqrt(jnp.sum(p1 * p1, axis=1, keepdims=True))
    proj = jnp.sum(p2 * e1, axis=1, keepdims=True)
    u2 = p2 - proj * e1
    e2 = u2 / jnp.sqrt(jnp.sum(u2 * u2, axis=1, keepdims=True))
    e3 = jnp.concatenate([
        e1[:, 1:2] * e2[:, 2:3] - e1[:, 2:3] * e2[:, 1:2],
        e1[:, 2:3] * e2[:, 0:1] - e1[:, 0:1] * e2[:, 2:3],
        e1[:, 0:1] * e2[:, 1:2] - e1[:, 1:2] * e2[:, 0:1],
    ], axis=1)

    c0 = c0_ref[...]
    c1 = c1_ref[...]
    c2 = c2_ref[...]
    for e, o_ref in ((e1, o0_ref), (e2, o1_ref), (e3, o2_ref)):
        o_ref[...] = e[:, 0:1] * c0 + e[:, 1:2] * c1 + e[:, 2:3] * c2


@jax.jit
def kernel(coeffs, pos, atomic_numbers):
    heavy = atomic_numbers != 1
    penalty = jnp.where(heavy, 0.0, jnp.inf).astype(jnp.float32)
    colbase = (jnp.sum(pos * pos, axis=1) + penalty)[None, :]
    posT2 = (2.0 * pos).T
    iota = jnp.arange(N, dtype=jnp.float32)[None, :]
    tab = jnp.concatenate([jnp.ones((N, 1), jnp.float32),
                           jnp.arange(N, dtype=jnp.float32)[:, None],
                           pos], axis=1)
    anum = atomic_numbers[:, None]
    c0 = coeffs[:, :, 0]
    c1 = coeffs[:, :, 1]
    c2 = coeffs[:, :, 2]

    grid = (N // R,)
    full = lambda *dims: pl.BlockSpec(dims, lambda i: (0,) * len(dims))
    rows = lambda *dims: pl.BlockSpec((R,) + dims, lambda i: (i,) + (0,) * len(dims))

    o0, o1, o2 = pl.pallas_call(
        _body,
        grid=grid,
        in_specs=[
            full(3, N),
            full(1, N),
            full(1, N),
            rows(1),
            rows(64), rows(64), rows(64),
            full(N, 5),
        ],
        out_specs=[rows(64), rows(64), rows(64)],
        out_shape=[jax.ShapeDtypeStruct((N, 64), jnp.float32)] * 3,
    )(posT2, colbase, iota, anum, c0, c1, c2, tab)
    return jnp.stack([o0, o1, o2], axis=-1)
